# baseline (speedup 1.0000x reference)
import functools

import jax
import jax.numpy as jnp
from jax import lax
from jax.experimental import pallas as pl
from jax.experimental.pallas import tpu as pltpu
from jax.experimental.pallas import tpu_sc as plsc

NC = 2
NS = 16
NW = NC * NS
L = 16

D_MODEL = 64
D_PAD = 128
SCALE = 8.0

CHUNK = 128
HALF = CHUNK // 2
NBUF = 5
AHEAD = 2

TC_ROWS = 1024


def _make_gather_kernel(n_idx: int):
    n_half = n_idx // 2
    assert n_idx % (NW * CHUNK) == 0
    per_w = n_half // NW
    nch = per_w // HALF
    assert nch % NBUF == 0
    ngrp = nch // NBUF

    mesh = plsc.VectorSubcoreMesh(
        core_axis_name="c", subcore_axis_name="s",
        num_cores=NC, num_subcores=NS,
    )

    scratch = [pltpu.VMEM((nch, CHUNK), jnp.int32)]
    scratch += [pltpu.VMEM((CHUNK, D_PAD), jnp.float32) for _ in range(NBUF)]
    scratch += [pltpu.SemaphoreType.DMA for _ in range(2 * NBUF)]

    @functools.partial(
        pl.kernel,
        out_type=jax.ShapeDtypeStruct((n_half, D_PAD), jnp.float32),
        mesh=mesh,
        scratch_types=scratch,
        compiler_params=pltpu.CompilerParams(use_tc_tiling_on_sc=True),
    )
    def emb(idx_hbm, table_hbm, out_hbm, idx_v, *rest):
        rows = rest[:NBUF]
        sem_in = rest[NBUF:2 * NBUF]
        sem_out = rest[2 * NBUF:]

        wid = lax.axis_index("s") * NC + lax.axis_index("c")
        base = wid * per_w

        pltpu.sync_copy(idx_hbm.at[wid], idx_v)

        def fire_gather(f, bf):
            pltpu.async_copy(table_hbm.at[idx_v.at[f]], rows[bf], sem_in[bf])

        def drain_gather(bf):
            pltpu.make_async_copy(
                table_hbm.at[pl.ds(0, CHUNK)], rows[bf], sem_in[bf]).wait()

        def drain_store(bf):
            pltpu.make_async_copy(
                rows[bf].at[pl.ds(0, HALF)],
                out_hbm.at[pl.ds(0, HALF)], sem_out[bf]).wait()

        for b in range(AHEAD):
            fire_gather(b, b)

        def group(i, carry):
            for b in range(NBUF):
                g = i * NBUF + b
                r = rows[b]
                drain_gather(b)

                def pack_row(j, c, r=r):
                    for k in range(D_MODEL // L):
                        r[j, pl.ds(D_MODEL + k * L, L)] = (
                            r[j + HALF, pl.ds(k * L, L)])
                    return c
                lax.fori_loop(0, HALF, pack_row, 0, unroll=4)

                pltpu.async_copy(
                    r.at[pl.ds(0, HALF)],
                    out_hbm.at[pl.ds(base + g * HALF, HALF)], sem_out[b])

                bf = (b + AHEAD) % NBUF
                if b + AHEAD < NBUF:
                    @pl.when(i > 0)
                    def _drain(bf=bf):
                        drain_store(bf)
                    fire_gather(g + AHEAD, bf)
                else:
                    drain_store(bf)

                    @pl.when(i + 1 < ngrp)
                    def _fire(g=g, bf=bf):
                        fire_gather(g + AHEAD, bf)
            return carry

        lax.fori_loop(0, ngrp, group, 0)

        for c in range(nch - (NBUF - AHEAD), nch):
            drain_store(c % NBUF)

    return emb


def _tc_unpack(packed, n_half):
    nblk = n_half // TC_ROWS

    def body(x_ref, o_ref):
        x = x_ref[...]
        o_ref[0] = x[:, :D_MODEL] * SCALE
        o_ref[1] = x[:, D_MODEL:] * SCALE

    return pl.pallas_call(
        body,
        grid=(nblk,),
        in_specs=[pl.BlockSpec((TC_ROWS, D_PAD), lambda n: (n, 0))],
        out_specs=pl.BlockSpec((2, TC_ROWS, D_MODEL), lambda n: (0, n, 0)),
        out_shape=jax.ShapeDtypeStruct((2, n_half, D_MODEL), jnp.float32),
    )(packed)


@functools.lru_cache(maxsize=None)
def _get_gather(n_idx: int):
    return _make_gather_kernel(n_idx)


def kernel(x, table):
    n_rows, n_cols = x.shape
    n_idx = n_rows * n_cols
    n_half = n_idx // 2
    flat = x.astype(jnp.int32).reshape(n_idx)
    h0 = flat[:n_half].reshape(NW, n_half // (NW * HALF), HALF)
    h1 = flat[n_half:].reshape(NW, n_half // (NW * HALF), HALF)
    idx = jnp.concatenate([h0, h1], axis=2)
    tablep = jnp.pad(table, ((0, 0), (0, D_PAD - D_MODEL)))
    packed = _get_gather(n_idx)(idx, tablep)
    out = _tc_unpack(packed, n_half)
    return out.reshape(n_rows, n_cols, D_MODEL)
